# baseline (speedup 1.0000x reference)
import jax
import jax.numpy as jnp
from jax.experimental import pallas as pl
from jax.experimental.pallas import tpu as pltpu

EMBED_DIM = 256
NUM_PTM = 26
NUM_PER = 128
NUM_EMB = NUM_PTM * NUM_PER
TEMP = 0.07
NEG = -1e9

BT = 256
NB = 8192 // BT
NW = NB + NUM_PTM - 2 + 9


def _dispatch_kernel(wl_ref, qs_ref, x_ref, emb_ref, q_ref, idx_ref):
    w = pl.program_id(0)
    t = wl_ref[1, w]
    fst = wl_ref[2, w]
    vld = wl_ref[3, w]
    x = x_ref[...]
    esl = emb_ref[pl.ds(t * NUM_PER, NUM_PER), :]
    logits = jax.lax.dot_general(
        x, esl, (((1,), (1,)), ((), ())), preferred_element_type=jnp.float32
    )
    rowmax = jnp.max(logits, axis=1, keepdims=True)
    e = jnp.exp(logits - rowmax)
    sim = e / jnp.sum(e, axis=1, keepdims=True)
    active = (qs_ref[...] == t) & (vld > 0)
    simz = jnp.where(active, sim, 0.0)
    contrib = jax.lax.dot_general(
        simz, esl, (((1,), (0,)), ((), ())), preferred_element_type=jnp.float32
    )
    col = jax.lax.broadcasted_iota(jnp.int32, logits.shape, 1)
    local = jnp.min(
        jnp.where(logits == rowmax, col, NUM_PER), axis=1, keepdims=True
    )
    idxg = t * NUM_PER + local

    @pl.when(fst == 1)
    def _():
        q_ref[...] = contrib
        idx_ref[...] = jnp.where(active, idxg, 0)

    @pl.when(fst == 0)
    def _():
        q_ref[...] += contrib
        idx_ref[...] = jnp.where(active, idxg, idx_ref[...])


def _unif_kernel(emb_ref, rinv_ref, cinv_ref, out_ref):
    b = pl.program_id(0)
    emb_blk = emb_ref[pl.ds(b * NUM_PER, NUM_PER), :]
    emb = emb_ref[...]
    sim = jax.lax.dot_general(
        emb_blk, emb, (((1,), (1,)), ((), ())), preferred_element_type=jnp.float32
    )
    rinv = rinv_ref[pl.ds(b * NUM_PER, NUM_PER), :]
    sim = sim * rinv * cinv_ref[...]
    row_g = b * NUM_PER + jax.lax.broadcasted_iota(jnp.int32, sim.shape, 0)
    col = jax.lax.broadcasted_iota(jnp.int32, sim.shape, 1)
    sim = jnp.where(col == row_g, NEG, sim)
    e = jnp.exp(sim * (1.0 / TEMP))
    sum_exp = jnp.sum(e, axis=1)
    pos_mask = (col >= b * NUM_PER) & (col < (b + 1) * NUM_PER)
    pos_sum = jnp.sum(jnp.where(pos_mask, e, 0.0), axis=1)
    part = jnp.sum(jnp.log(pos_sum) - jnp.log(sum_exp))

    @pl.when(b == 0)
    def _():
        out_ref[0, 0] = 0.0

    out_ref[0, 0] += part


def _build_worklist(qs):
    qsb = qs.reshape(NB, BT)
    counts = jnp.sum(
        qsb[:, :, None] == jnp.arange(NUM_PTM, dtype=jnp.int32)[None, None, :],
        axis=1,
    )
    exists = (counts > 0).reshape(-1)
    pos = jnp.cumsum(exists) - 1
    total = pos[-1] + 1
    tgt = jnp.where(exists, pos, NW)
    blk_flat = jnp.repeat(jnp.arange(NB, dtype=jnp.int32), NUM_PTM)
    t_flat = jnp.tile(jnp.arange(NUM_PTM, dtype=jnp.int32), NB)
    zeros = jnp.zeros((NW,), jnp.int32)
    wl_blk = zeros.at[tgt].set(blk_flat, mode="drop")
    wl_t = zeros.at[tgt].set(t_flat, mode="drop")
    wl_blk = jax.lax.cummax(wl_blk)
    valid = (jnp.arange(NW) < total).astype(jnp.int32)
    wl_t = jnp.where(valid == 1, wl_t, wl_t[total - 1])
    prev_blk = jnp.concatenate([jnp.full((1,), -1, jnp.int32), wl_blk[:-1]])
    fst = (wl_blk != prev_blk).astype(jnp.int32)
    return jnp.stack([wl_blk, wl_t, fst, valid])


def kernel(x, Q, embeddings):
    n_rows = x.shape[0]
    qi = Q.astype(jnp.int32)
    sort_idx = jnp.argsort(qi)
    qs = qi[sort_idx]
    xs = x[sort_idx]
    wl = _build_worklist(qs)

    q_srt, idx_srt = pl.pallas_call(
        _dispatch_kernel,
        grid_spec=pltpu.PrefetchScalarGridSpec(
            num_scalar_prefetch=1,
            grid=(NW,),
            in_specs=[
                pl.BlockSpec((BT, 1), lambda w, wl: (wl[0, w], 0)),
                pl.BlockSpec((BT, EMBED_DIM), lambda w, wl: (wl[0, w], 0)),
                pl.BlockSpec((NUM_EMB, EMBED_DIM), lambda w, wl: (0, 0)),
            ],
            out_specs=[
                pl.BlockSpec((BT, EMBED_DIM), lambda w, wl: (wl[0, w], 0)),
                pl.BlockSpec((BT, 1), lambda w, wl: (wl[0, w], 0)),
            ],
        ),
        out_shape=[
            jax.ShapeDtypeStruct((n_rows, EMBED_DIM), jnp.float32),
            jax.ShapeDtypeStruct((n_rows, 1), jnp.int32),
        ],
    )(wl, qs.reshape(-1, 1), xs, embeddings)

    quantized = jnp.zeros((n_rows, EMBED_DIM), jnp.float32).at[sort_idx].set(q_srt)
    idx = jnp.zeros((n_rows,), jnp.int32).at[sort_idx].set(idx_srt.reshape(-1))

    norms = jnp.sqrt(jnp.sum(embeddings * embeddings, axis=1))
    rinv = (1.0 / norms).reshape(-1, 1)
    cinv = (1.0 / norms).reshape(1, -1)

    total = pl.pallas_call(
        _unif_kernel,
        grid=(NUM_PTM,),
        in_specs=[
            pl.BlockSpec((NUM_EMB, EMBED_DIM), lambda b: (0, 0)),
            pl.BlockSpec((NUM_EMB, 1), lambda b: (0, 0)),
            pl.BlockSpec((1, NUM_EMB), lambda b: (0, 0)),
        ],
        out_specs=pl.BlockSpec(memory_space=pltpu.SMEM),
        out_shape=jax.ShapeDtypeStruct((1, 1), jnp.float32),
    )(embeddings, rinv, cinv)

    uniform_loss = -(total[0, 0] / NUM_EMB)
    loss = jnp.float32(0.0)
    return quantized, loss, uniform_loss, idx


# speedup vs baseline: 1.7327x; 1.7327x over previous
import jax
import jax.numpy as jnp
from jax.experimental import pallas as pl
from jax.experimental.pallas import tpu as pltpu

EMBED_DIM = 256
NUM_PTM = 26
NUM_PER = 128
NUM_EMB = NUM_PTM * NUM_PER
TEMP = 0.07
NEG = -1e9

BR = 512


def _quant_kernel(q_in_ref, x_ref, emb_ref, q_ref, idx_ref):
    x = x_ref[...]
    emb = emb_ref[...]
    logits = jax.lax.dot_general(
        x, emb, (((1,), (1,)), ((), ())), preferred_element_type=jnp.float32
    )
    qt = q_in_ref[...]
    lsel = jnp.zeros((x.shape[0], NUM_PER), jnp.float32)
    for t in range(NUM_PTM):
        piece = logits[:, t * NUM_PER:(t + 1) * NUM_PER]
        lsel = jnp.where(qt == t, piece, lsel)
    rowmax = jnp.max(lsel, axis=1, keepdims=True)
    e = jnp.exp(lsel - rowmax)
    s = jnp.sum(e, axis=1, keepdims=True)
    eb = e.astype(jnp.bfloat16)
    zero = jnp.zeros_like(eb)
    ez = jnp.concatenate(
        [jnp.where(qt == t, eb, zero) for t in range(NUM_PTM)], axis=1
    )
    q_un = jax.lax.dot_general(
        ez, emb, (((1,), (0,)), ((), ())), preferred_element_type=jnp.float32
    )
    q_ref[...] = q_un / s
    col = jax.lax.broadcasted_iota(jnp.int32, e.shape, 1)
    local = jnp.min(
        jnp.where(lsel == rowmax, col, NUM_PER), axis=1, keepdims=True
    )
    idx_ref[...] = qt * NUM_PER + local


def _unif_kernel(emb_ref, rinv_ref, cinv_ref, out_ref):
    b = pl.program_id(0)
    emb_blk = emb_ref[pl.ds(b * NUM_PER, NUM_PER), :]
    emb = emb_ref[...]
    sim = jax.lax.dot_general(
        emb_blk, emb, (((1,), (1,)), ((), ())), preferred_element_type=jnp.float32
    )
    rinv = rinv_ref[pl.ds(b * NUM_PER, NUM_PER), :]
    sim = sim * rinv * cinv_ref[...]
    row_g = b * NUM_PER + jax.lax.broadcasted_iota(jnp.int32, sim.shape, 0)
    col = jax.lax.broadcasted_iota(jnp.int32, sim.shape, 1)
    sim = jnp.where(col == row_g, NEG, sim)
    e = jnp.exp(sim * (1.0 / TEMP))
    sum_exp = jnp.sum(e, axis=1)
    pos_mask = (col >= b * NUM_PER) & (col < (b + 1) * NUM_PER)
    pos_sum = jnp.sum(jnp.where(pos_mask, e, 0.0), axis=1)
    part = jnp.sum(jnp.log(pos_sum) - jnp.log(sum_exp))

    @pl.when(b == 0)
    def _():
        out_ref[0, 0] = 0.0

    out_ref[0, 0] += part


def kernel(x, Q, embeddings):
    n_rows = x.shape[0]
    grid = n_rows // BR
    qcol = Q.astype(jnp.int32).reshape(-1, 1)

    quantized, idx = pl.pallas_call(
        _quant_kernel,
        grid=(grid,),
        in_specs=[
            pl.BlockSpec((BR, 1), lambda i: (i, 0)),
            pl.BlockSpec((BR, EMBED_DIM), lambda i: (i, 0)),
            pl.BlockSpec((NUM_EMB, EMBED_DIM), lambda i: (0, 0)),
        ],
        out_specs=[
            pl.BlockSpec((BR, EMBED_DIM), lambda i: (i, 0)),
            pl.BlockSpec((BR, 1), lambda i: (i, 0)),
        ],
        out_shape=[
            jax.ShapeDtypeStruct((n_rows, EMBED_DIM), jnp.float32),
            jax.ShapeDtypeStruct((n_rows, 1), jnp.int32),
        ],
    )(qcol, x.astype(jnp.bfloat16), embeddings.astype(jnp.bfloat16))

    norms = jnp.sqrt(jnp.sum(embeddings * embeddings, axis=1))
    rinv = (1.0 / norms).reshape(-1, 1)
    cinv = (1.0 / norms).reshape(1, -1)

    total = pl.pallas_call(
        _unif_kernel,
        grid=(NUM_PTM,),
        in_specs=[
            pl.BlockSpec((NUM_EMB, EMBED_DIM), lambda b: (0, 0)),
            pl.BlockSpec((NUM_EMB, 1), lambda b: (0, 0)),
            pl.BlockSpec((1, NUM_EMB), lambda b: (0, 0)),
        ],
        out_specs=pl.BlockSpec(memory_space=pltpu.SMEM),
        out_shape=jax.ShapeDtypeStruct((1, 1), jnp.float32),
    )(embeddings, rinv, cinv)

    uniform_loss = -(total[0, 0] / NUM_EMB)
    loss = jnp.float32(0.0)
    return quantized, loss, uniform_loss, idx.reshape(-1)


# speedup vs baseline: 2.1430x; 1.2368x over previous
import jax
import jax.numpy as jnp
from jax.experimental import pallas as pl
from jax.experimental.pallas import tpu as pltpu

EMBED_DIM = 256
NUM_PTM = 26
NUM_PER = 128
NUM_EMB = NUM_PTM * NUM_PER
TEMP = 0.07
NEG = -1e9

BR = 512
BU = 256


def _quant_kernel(q_in_ref, x_ref, emb_ref, q_ref, idx_ref):
    x = x_ref[...]
    emb = emb_ref[...]
    logits = jax.lax.dot_general(
        x, emb, (((1,), (1,)), ((), ())), preferred_element_type=jnp.float32
    )
    qt = q_in_ref[...]
    lsel = jnp.zeros((x.shape[0], NUM_PER), jnp.float32)
    for t in range(NUM_PTM):
        piece = logits[:, t * NUM_PER:(t + 1) * NUM_PER]
        lsel = jnp.where(qt == t, piece, lsel)
    rowmax = jnp.max(lsel, axis=1, keepdims=True)
    e = jnp.exp(lsel - rowmax)
    s = jnp.sum(e, axis=1, keepdims=True)
    eb = e.astype(jnp.bfloat16)
    zero = jnp.zeros_like(eb)
    ez = jnp.concatenate(
        [jnp.where(qt == t, eb, zero) for t in range(NUM_PTM)], axis=1
    )
    q_un = jax.lax.dot_general(
        ez, emb, (((1,), (0,)), ((), ())), preferred_element_type=jnp.float32
    )
    q_ref[...] = q_un / s
    col = jax.lax.broadcasted_iota(jnp.int32, e.shape, 1)
    local = jnp.min(
        jnp.where(lsel == rowmax, col, NUM_PER), axis=1, keepdims=True
    )
    idx_ref[...] = qt * NUM_PER + local


def _unif_kernel(emb_ref, out_ref):
    b = pl.program_id(0)
    emb_blk = emb_ref[pl.ds(b * BU, BU), :]
    emb = emb_ref[...]
    sim = jax.lax.dot_general(
        emb_blk, emb, (((1,), (1,)), ((), ())), preferred_element_type=jnp.float32
    )
    e = jnp.exp(sim * (1.0 / TEMP))
    row_sum = jnp.sum(e, axis=1, keepdims=True)
    ps = jax.lax.dot_general(
        emb_blk, emb_blk, (((1,), (1,)), ((), ())),
        preferred_element_type=jnp.float32,
    )
    r0 = jax.lax.broadcasted_iota(jnp.int32, ps.shape, 0)
    c0 = jax.lax.broadcasted_iota(jnp.int32, ps.shape, 1)
    pos_mask = (c0 // NUM_PER) == (r0 // NUM_PER)
    pe = jnp.exp(ps * (1.0 / TEMP))
    diag = jnp.sum(jnp.where(r0 == c0, pe, 0.0), axis=1, keepdims=True)
    pos_sum = jnp.sum(jnp.where(pos_mask, pe, 0.0), axis=1, keepdims=True) - diag
    sum_exp = row_sum - diag
    part = jnp.sum(jnp.log(pos_sum) - jnp.log(sum_exp))

    @pl.when(b == 0)
    def _():
        out_ref[0, 0] = 0.0

    out_ref[0, 0] += part


def kernel(x, Q, embeddings):
    n_rows = x.shape[0]
    grid = n_rows // BR
    qcol = Q.astype(jnp.int32).reshape(-1, 1)

    quantized, idx = pl.pallas_call(
        _quant_kernel,
        grid=(grid,),
        in_specs=[
            pl.BlockSpec((BR, 1), lambda i: (i, 0)),
            pl.BlockSpec((BR, EMBED_DIM), lambda i: (i, 0)),
            pl.BlockSpec((NUM_EMB, EMBED_DIM), lambda i: (0, 0)),
        ],
        out_specs=[
            pl.BlockSpec((BR, EMBED_DIM), lambda i: (i, 0)),
            pl.BlockSpec((BR, 1), lambda i: (i, 0)),
        ],
        out_shape=[
            jax.ShapeDtypeStruct((n_rows, EMBED_DIM), jnp.float32),
            jax.ShapeDtypeStruct((n_rows, 1), jnp.int32),
        ],
    )(qcol, x.astype(jnp.bfloat16), embeddings.astype(jnp.bfloat16))

    total = pl.pallas_call(
        _unif_kernel,
        grid=(NUM_EMB // BU,),
        in_specs=[pl.BlockSpec((NUM_EMB, EMBED_DIM), lambda b: (0, 0))],
        out_specs=pl.BlockSpec(memory_space=pltpu.SMEM),
        out_shape=jax.ShapeDtypeStruct((1, 1), jnp.float32),
    )(embeddings)

    uniform_loss = -(total[0, 0] / NUM_EMB)
    loss = jnp.float32(0.0)
    return quantized, loss, uniform_loss, idx.reshape(-1)


# speedup vs baseline: 2.1634x; 1.0095x over previous
import jax
import jax.numpy as jnp
from jax.experimental import pallas as pl
from jax.experimental.pallas import tpu as pltpu

EMBED_DIM = 256
NUM_PTM = 26
NUM_PER = 128
NUM_EMB = NUM_PTM * NUM_PER
TEMP = 0.07
NEG = -1e9

BR = 512
BU = 256
NU = NUM_EMB // BU


def _fused_kernel(q_in_ref, x_ref, emb_ref, emb32_ref, q_ref, idx_ref, out_ref):
    i = pl.program_id(0)
    x = x_ref[...]
    emb = emb_ref[...]
    logits = jax.lax.dot_general(
        x, emb, (((1,), (1,)), ((), ())), preferred_element_type=jnp.float32
    )
    qt = q_in_ref[...]
    lsel = jnp.zeros((x.shape[0], NUM_PER), jnp.float32)
    for t in range(NUM_PTM):
        piece = logits[:, t * NUM_PER:(t + 1) * NUM_PER]
        lsel = jnp.where(qt == t, piece, lsel)
    rowmax = jnp.max(lsel, axis=1, keepdims=True)
    e = jnp.exp(lsel - rowmax)
    s = jnp.sum(e, axis=1, keepdims=True)
    eb = e.astype(jnp.bfloat16)
    zero = jnp.zeros_like(eb)
    ez = jnp.concatenate(
        [jnp.where(qt == t, eb, zero) for t in range(NUM_PTM)], axis=1
    )
    q_un = jax.lax.dot_general(
        ez, emb, (((1,), (0,)), ((), ())), preferred_element_type=jnp.float32
    )
    q_ref[...] = q_un / s
    col = jax.lax.broadcasted_iota(jnp.int32, e.shape, 1)
    local = jnp.min(
        jnp.where(lsel == rowmax, col, NUM_PER), axis=1, keepdims=True
    )
    idx_ref[...] = qt * NUM_PER + local

    @pl.when(i == 0)
    def _():
        out_ref[0, 0] = 0.0

    @pl.when(i < NU)
    def _():
        emb32 = emb32_ref[...]
        emb_blk = emb32_ref[pl.ds(i * BU, BU), :]
        sim = jax.lax.dot_general(
            emb_blk, emb32, (((1,), (1,)), ((), ())),
            preferred_element_type=jnp.float32,
        )
        ev = jnp.exp(sim * (1.0 / TEMP))
        row_sum = jnp.sum(ev, axis=1, keepdims=True)
        ps = jax.lax.dot_general(
            emb_blk, emb_blk, (((1,), (1,)), ((), ())),
            preferred_element_type=jnp.float32,
        )
        r0 = jax.lax.broadcasted_iota(jnp.int32, ps.shape, 0)
        c0 = jax.lax.broadcasted_iota(jnp.int32, ps.shape, 1)
        pos_mask = (c0 // NUM_PER) == (r0 // NUM_PER)
        pe = jnp.exp(ps * (1.0 / TEMP))
        diag = jnp.sum(jnp.where(r0 == c0, pe, 0.0), axis=1, keepdims=True)
        pos_sum = (
            jnp.sum(jnp.where(pos_mask, pe, 0.0), axis=1, keepdims=True) - diag
        )
        sum_exp = row_sum - diag
        out_ref[0, 0] += jnp.sum(jnp.log(pos_sum) - jnp.log(sum_exp))


def kernel(x, Q, embeddings):
    n_rows = x.shape[0]
    grid = n_rows // BR
    qcol = Q.astype(jnp.int32).reshape(-1, 1)

    quantized, idx, total = pl.pallas_call(
        _fused_kernel,
        grid=(grid,),
        in_specs=[
            pl.BlockSpec((BR, 1), lambda i: (i, 0)),
            pl.BlockSpec((BR, EMBED_DIM), lambda i: (i, 0)),
            pl.BlockSpec((NUM_EMB, EMBED_DIM), lambda i: (0, 0)),
            pl.BlockSpec((NUM_EMB, EMBED_DIM), lambda i: (0, 0)),
        ],
        out_specs=[
            pl.BlockSpec((BR, EMBED_DIM), lambda i: (i, 0)),
            pl.BlockSpec((BR, 1), lambda i: (i, 0)),
            pl.BlockSpec(memory_space=pltpu.SMEM),
        ],
        out_shape=[
            jax.ShapeDtypeStruct((n_rows, EMBED_DIM), jnp.float32),
            jax.ShapeDtypeStruct((n_rows, 1), jnp.int32),
            jax.ShapeDtypeStruct((1, 1), jnp.float32),
        ],
    )(qcol, x.astype(jnp.bfloat16), embeddings.astype(jnp.bfloat16), embeddings)

    uniform_loss = -(total[0, 0] / NUM_EMB)
    loss = jnp.float32(0.0)
    return quantized, loss, uniform_loss, idx.reshape(-1)


# speedup vs baseline: 2.3626x; 1.0921x over previous
import jax
import jax.numpy as jnp
from jax.experimental import pallas as pl
from jax.experimental.pallas import tpu as pltpu

EMBED_DIM = 256
NUM_PTM = 26
NUM_PER = 128
NUM_EMB = NUM_PTM * NUM_PER
TEMP = 0.07
NEG = -1e9

BR = 1024
BU = 256
NU = NUM_EMB // BU


def _fused_kernel(q_in_ref, x_ref, emb_ref, emb32_ref, q_ref, idx_ref, out_ref):
    i = pl.program_id(0)
    x = x_ref[...]
    emb = emb_ref[...]
    logits = jax.lax.dot_general(
        x, emb, (((1,), (1,)), ((), ())), preferred_element_type=jnp.float32
    )
    qt = q_in_ref[...]
    RC = 128
    lsel_chunks = []
    for r in range(0, BR, RC):
        qt_c = qt[r:r + RC]
        lsel_c = jnp.zeros((RC, NUM_PER), jnp.float32)
        for t in range(NUM_PTM):
            piece = logits[r:r + RC, t * NUM_PER:(t + 1) * NUM_PER]
            lsel_c = jnp.where(qt_c == t, piece, lsel_c)
        lsel_chunks.append(lsel_c)
    lsel = jnp.concatenate(lsel_chunks, axis=0)
    rowmax = jnp.max(lsel, axis=1, keepdims=True)
    e = jnp.exp(lsel - rowmax)
    s = jnp.sum(e, axis=1, keepdims=True)
    eb = e.astype(jnp.bfloat16)
    ez_chunks = []
    for r in range(0, BR, RC):
        qt_c = qt[r:r + RC]
        eb_c = eb[r:r + RC]
        zero = jnp.zeros_like(eb_c)
        ez_chunks.append(jnp.concatenate(
            [jnp.where(qt_c == t, eb_c, zero) for t in range(NUM_PTM)], axis=1
        ))
    ez = jnp.concatenate(ez_chunks, axis=0)
    q_un = jax.lax.dot_general(
        ez, emb, (((1,), (0,)), ((), ())), preferred_element_type=jnp.float32
    )
    q_ref[...] = q_un / s
    col = jax.lax.broadcasted_iota(jnp.int32, e.shape, 1)
    local = jnp.min(
        jnp.where(lsel == rowmax, col, NUM_PER), axis=1, keepdims=True
    )
    idx_ref[...] = qt * NUM_PER + local

    @pl.when(i == 0)
    def _():
        out_ref[0, 0] = 0.0

    @pl.when(i < NU)
    def _():
        emb32 = emb32_ref[...]
        emb_blk = emb32_ref[pl.ds(i * BU, BU), :]
        sim = jax.lax.dot_general(
            emb_blk, emb32, (((1,), (1,)), ((), ())),
            preferred_element_type=jnp.float32,
        )
        ev = jnp.exp(sim * (1.0 / TEMP))
        row_sum = jnp.sum(ev, axis=1, keepdims=True)
        ps = jax.lax.dot_general(
            emb_blk, emb_blk, (((1,), (1,)), ((), ())),
            preferred_element_type=jnp.float32,
        )
        r0 = jax.lax.broadcasted_iota(jnp.int32, ps.shape, 0)
        c0 = jax.lax.broadcasted_iota(jnp.int32, ps.shape, 1)
        pos_mask = (c0 // NUM_PER) == (r0 // NUM_PER)
        pe = jnp.exp(ps * (1.0 / TEMP))
        diag = jnp.sum(jnp.where(r0 == c0, pe, 0.0), axis=1, keepdims=True)
        pos_sum = (
            jnp.sum(jnp.where(pos_mask, pe, 0.0), axis=1, keepdims=True) - diag
        )
        sum_exp = row_sum - diag
        out_ref[0, 0] += jnp.sum(jnp.log(pos_sum) - jnp.log(sum_exp))


def kernel(x, Q, embeddings):
    n_rows = x.shape[0]
    grid = n_rows // BR
    qcol = Q.astype(jnp.int32).reshape(-1, 1)

    quantized, idx, total = pl.pallas_call(
        _fused_kernel,
        grid=(grid,),
        in_specs=[
            pl.BlockSpec((BR, 1), lambda i: (i, 0)),
            pl.BlockSpec((BR, EMBED_DIM), lambda i: (i, 0)),
            pl.BlockSpec((NUM_EMB, EMBED_DIM), lambda i: (0, 0)),
            pl.BlockSpec((NUM_EMB, EMBED_DIM), lambda i: (0, 0)),
        ],
        out_specs=[
            pl.BlockSpec((BR, EMBED_DIM), lambda i: (i, 0)),
            pl.BlockSpec((BR, 1), lambda i: (i, 0)),
            pl.BlockSpec(memory_space=pltpu.SMEM),
        ],
        out_shape=[
            jax.ShapeDtypeStruct((n_rows, EMBED_DIM), jnp.float32),
            jax.ShapeDtypeStruct((n_rows, 1), jnp.int32),
            jax.ShapeDtypeStruct((1, 1), jnp.float32),
        ],
    )(qcol, x.astype(jnp.bfloat16), embeddings.astype(jnp.bfloat16), embeddings)

    uniform_loss = -(total[0, 0] / NUM_EMB)
    loss = jnp.float32(0.0)
    return quantized, loss, uniform_loss, idx.reshape(-1)
